# baseline (speedup 1.0000x reference)
import functools

import jax
import jax.numpy as jnp
from jax import lax
from jax.experimental import pallas as pl
from jax.experimental.pallas import tpu as pltpu
from jax.experimental.pallas import tpu_sc as plsc

_B, _D, _H, _A = 16384, 500, 500, 8
_BT = 1024
_NC, _NS, _L = 2, 16, 16
_NW = _NC * _NS
_RPW = _B // _NW


def _dense_body(xt_ref, w1_ref, b1_ref, w2t_ref, b2_ref, pdf_ref):
    xt = xt_ref[...]
    ht = jax.lax.dot_general(
        w1_ref[...], xt, (((0,), (0,)), ((), ())),
        preferred_element_type=jnp.float32)
    ht = jnp.maximum(ht + b1_ref[...], 0.0)
    zt = jnp.dot(w2t_ref[...], ht,
                 preferred_element_type=jnp.float32)
    zt = zt + b2_ref[...]
    m = jnp.max(zt, axis=0, keepdims=True)
    e = jnp.exp(zt - m)
    prob = e / jnp.sum(e, axis=0, keepdims=True)
    logits = jnp.log(prob + 1e-20)
    m2 = jnp.max(logits, axis=0, keepdims=True)
    pdf_ref[...] = jnp.exp(logits - m2)


_sc_mesh = plsc.VectorSubcoreMesh(core_axis_name="c", subcore_axis_name="s")


@functools.partial(
    pl.kernel,
    mesh=_sc_mesh,
    out_type=jax.ShapeDtypeStruct((_B,), jnp.int32),
    scratch_types=[
        pltpu.VMEM((_A, _RPW), jnp.float32),
        pltpu.VMEM((_RPW,), jnp.float32),
        pltpu.VMEM((_RPW,), jnp.int32),
    ],
)
def _sample_sc(pdf_hbm, ut_hbm, out_hbm, pdf_v, u_v, out_v):
    wid = lax.axis_index("s") * _NC + lax.axis_index("c")
    base = wid * _RPW
    for a in range(_A):
        pltpu.sync_copy(pdf_hbm.at[a, pl.ds(base, _RPW)], pdf_v.at[a])
    pltpu.sync_copy(ut_hbm.at[0, pl.ds(base, _RPW)], u_v)
    for j in range(_RPW // _L):
        sl = pl.ds(j * _L, _L)
        run = pdf_v[0, sl]
        cdfs = [run]
        for a in range(1, _A):
            run = run + pdf_v[a, sl]
            cdfs.append(run)
        us = u_v[sl] * run
        cnt = jnp.where(cdfs[0] <= us, 1.0, 0.0)
        for a in range(1, _A):
            cnt = cnt + jnp.where(cdfs[a] <= us, 1.0, 0.0)
        out_v[sl] = cnt.astype(jnp.int32)
    pltpu.sync_copy(out_v, out_hbm.at[pl.ds(base, _RPW)])


def kernel(inputs, u, W1, b1, W2, b2):
    xt = inputs.T
    ut = u.T
    w2t = W2.T
    b1r = b1.reshape(_H, 1)
    b2r = b2.reshape(_A, 1)
    grid = (_B // _BT,)
    pdf = pl.pallas_call(
        _dense_body,
        grid=grid,
        in_specs=[
            pl.BlockSpec((_D, _BT), lambda i: (0, i)),
            pl.BlockSpec((_D, _H), lambda i: (0, 0)),
            pl.BlockSpec((_H, 1), lambda i: (0, 0)),
            pl.BlockSpec((_A, _D), lambda i: (0, 0)),
            pl.BlockSpec((_A, 1), lambda i: (0, 0)),
        ],
        out_specs=pl.BlockSpec((_A, _BT), lambda i: (0, i)),
        out_shape=jax.ShapeDtypeStruct((_A, _B), jnp.float32),
    )(xt, W1, b1r, w2t, b2r)
    samples = _sample_sc(pdf, ut)
    return samples.astype(jnp.int64)
